# baseline (speedup 1.0000x reference)
import jax
import jax.numpy as jnp
from jax.experimental import pallas as pl
from jax.experimental.pallas import tpu as pltpu

_ISSUE_UNROLL = 32
_TILE = 512


def _fused_kernel(Dw, Dp, ids_smem, w_hbm, pos_hbm, out_ref, sem_w, sem_w2, sem_p):
    tile = out_ref.shape[0]
    base = pl.program_id(0) * tile

    pcopy = pltpu.make_async_copy(
        pos_hbm.at[pl.ds(base, tile)],
        out_ref.at[:, pl.ds(Dw, Dp)],
        sem_p,
    )
    pcopy.start()

    unroll = _ISSUE_UNROLL if tile % _ISSUE_UNROLL == 0 else 1

    @pl.loop(0, tile // unroll)
    def _issue(c):
        r0 = c * unroll
        for u in range(unroll):
            r = r0 + u
            tok = ids_smem[base + r]
            pltpu.make_async_copy(
                w_hbm.at[tok],
                out_ref.at[r, pl.ds(0, Dw)],
                sem_w if u % 2 == 0 else sem_w2,
            ).start()

    half = tile // 2
    pltpu.make_async_copy(
        w_hbm.at[pl.ds(0, half)],
        out_ref.at[pl.ds(0, half), pl.ds(0, Dw)],
        sem_w,
    ).wait()
    pltpu.make_async_copy(
        w_hbm.at[pl.ds(0, half)],
        out_ref.at[pl.ds(0, half), pl.ds(0, Dw)],
        sem_w2,
    ).wait()
    pcopy.wait()


def _pick_tile(L):
    if L <= _TILE:
        return L
    for t in (_TILE, 512, 256, 128, 64, 32, 16, 8):
        if L % t == 0:
            return t
    return L


def kernel(ids, word_table, pos_table):
    L = ids.shape[0]
    V, Dw = word_table.shape
    P, Dp = pos_table.shape
    assert P >= L, "position table must cover the sequence length"

    ids = ids.astype(jnp.int32)
    pos_used = pos_table[:L]
    tile = _pick_tile(L)
    grid = (L // tile,)

    out = pl.pallas_call(
        lambda *refs: _fused_kernel(Dw, Dp, *refs),
        out_shape=jax.ShapeDtypeStruct((L, Dw + Dp), word_table.dtype),
        grid_spec=pltpu.PrefetchScalarGridSpec(
            num_scalar_prefetch=1,
            grid=grid,
            in_specs=[
                pl.BlockSpec(memory_space=pl.ANY),
                pl.BlockSpec(memory_space=pl.ANY),
            ],
            out_specs=pl.BlockSpec((tile, Dw + Dp), lambda i, ids_ref: (i, 0)),
            scratch_shapes=[pltpu.SemaphoreType.DMA(()),
                            pltpu.SemaphoreType.DMA(()),
                            pltpu.SemaphoreType.DMA(())],
        ),
        compiler_params=pltpu.CompilerParams(
            dimension_semantics=("parallel",),
            disable_bounds_checks=True,
        ),
    )(ids, word_table, pos_used)
    return out


# speedup vs baseline: 1.0455x; 1.0455x over previous
import jax
import jax.numpy as jnp
from jax.experimental import pallas as pl
from jax.experimental.pallas import tpu as pltpu

_ISSUE_UNROLL = 32
_NCHUNKS = 4


def _pipelined_kernel(Dw, Dp, rows, chunk,
                      ids_smem, w_hbm, pos_hbm, out_hbm, *scr):
    nchunks = rows // chunk
    bufs = scr[:nchunks]
    sems_w = scr[nchunks:2 * nchunks]
    sems_p = scr[2 * nchunks:3 * nchunks]
    sems_o = scr[3 * nchunks:4 * nchunks]
    base = pl.program_id(0) * rows

    unroll = _ISSUE_UNROLL if chunk % _ISSUE_UNROLL == 0 else 1

    def issue_chunk(c):
        start = base + c * chunk
        buf = bufs[c]
        pltpu.make_async_copy(
            pos_hbm.at[pl.ds(start, chunk)],
            buf.at[:, pl.ds(Dw, Dp)],
            sems_p[c],
        ).start()

        @pl.loop(0, chunk // unroll)
        def _issue(cc):
            r0 = cc * unroll
            for u in range(unroll):
                r = r0 + u
                tok = ids_smem[start + r]
                pltpu.make_async_copy(
                    w_hbm.at[tok],
                    buf.at[r, pl.ds(0, Dw)],
                    sems_w[c],
                ).start()

    def wait_chunk(c):
        pltpu.make_async_copy(
            w_hbm.at[pl.ds(0, chunk)],
            bufs[c].at[:, pl.ds(0, Dw)],
            sems_w[c],
        ).wait()
        pltpu.make_async_copy(
            pos_hbm.at[pl.ds(0, chunk)],
            bufs[c].at[:, pl.ds(Dw, Dp)],
            sems_p[c],
        ).wait()

    def start_out(c):
        pltpu.make_async_copy(
            bufs[c],
            out_hbm.at[pl.ds(base + c * chunk, chunk)],
            sems_o[c],
        ).start()

    for c in range(nchunks):
        issue_chunk(c)
        if c >= 1:
            wait_chunk(c - 1)
            start_out(c - 1)

    wait_chunk(nchunks - 1)
    start_out(nchunks - 1)
    for c in range(nchunks):
        pltpu.make_async_copy(
            bufs[c],
            out_hbm.at[pl.ds(base, chunk)],
            sems_o[c],
        ).wait()


def kernel(ids, word_table, pos_table):
    L = ids.shape[0]
    V, Dw = word_table.shape
    P, Dp = pos_table.shape
    assert P >= L, "position table must cover the sequence length"

    ids = ids.astype(jnp.int32)
    pos_used = pos_table[:L]
    ncores = 2 if L % 2 == 0 else 1
    rows = L // ncores
    nchunks = _NCHUNKS if rows % _NCHUNKS == 0 else 1
    chunk = rows // nchunks

    scratch = ([pltpu.VMEM((chunk, Dw + Dp), word_table.dtype)] * nchunks +
               [pltpu.SemaphoreType.DMA(())] * (3 * nchunks))

    out = pl.pallas_call(
        lambda *refs: _pipelined_kernel(Dw, Dp, rows, chunk, *refs),
        out_shape=jax.ShapeDtypeStruct((L, Dw + Dp), word_table.dtype),
        grid_spec=pltpu.PrefetchScalarGridSpec(
            num_scalar_prefetch=1,
            grid=(ncores,),
            in_specs=[
                pl.BlockSpec(memory_space=pl.ANY),
                pl.BlockSpec(memory_space=pl.ANY),
            ],
            out_specs=pl.BlockSpec(memory_space=pl.ANY),
            scratch_shapes=scratch,
        ),
        compiler_params=pltpu.CompilerParams(
            dimension_semantics=("parallel",),
            disable_bounds_checks=True,
        ),
    )(ids, word_table, pos_used)
    return out


# speedup vs baseline: 1.0541x; 1.0082x over previous
import jax
import jax.numpy as jnp
from jax.experimental import pallas as pl
from jax.experimental.pallas import tpu as pltpu

_ISSUE_UNROLL = 32
_TILE = 1024


def _fused_kernel(Dw, Dp, ids_smem, w_hbm, pos_hbm, out_ref, sem_w, sem_p):
    tile = out_ref.shape[0]
    base = pl.program_id(0) * tile

    pcopy = pltpu.make_async_copy(
        pos_hbm.at[pl.ds(base, tile)],
        out_ref.at[:, pl.ds(Dw, Dp)],
        sem_p,
    )
    pcopy.start()

    unroll = _ISSUE_UNROLL if tile % _ISSUE_UNROLL == 0 else 1

    @pl.loop(0, tile // unroll)
    def _issue(c):
        r0 = c * unroll
        for u in range(unroll):
            r = r0 + u
            tok = ids_smem[base + r]
            pltpu.make_async_copy(
                w_hbm.at[tok],
                out_ref.at[r, pl.ds(0, Dw)],
                sem_w,
            ).start()

    pltpu.make_async_copy(
        w_hbm.at[pl.ds(0, tile)],
        out_ref.at[:, pl.ds(0, Dw)],
        sem_w,
    ).wait()
    pcopy.wait()


def _pick_tile(L):
    if L <= _TILE:
        return L
    for t in (_TILE, 512, 256, 128, 64, 32, 16, 8):
        if L % t == 0:
            return t
    return L


def kernel(ids, word_table, pos_table):
    L = ids.shape[0]
    V, Dw = word_table.shape
    P, Dp = pos_table.shape
    assert P >= L, "position table must cover the sequence length"

    ids = ids.astype(jnp.int32)
    pos_used = pos_table[:L]
    tile = _pick_tile(L)
    grid = (L // tile,)

    out = pl.pallas_call(
        lambda *refs: _fused_kernel(Dw, Dp, *refs),
        out_shape=jax.ShapeDtypeStruct((L, Dw + Dp), word_table.dtype),
        grid_spec=pltpu.PrefetchScalarGridSpec(
            num_scalar_prefetch=1,
            grid=grid,
            in_specs=[
                pl.BlockSpec(memory_space=pl.ANY),
                pl.BlockSpec(memory_space=pl.ANY),
            ],
            out_specs=pl.BlockSpec((tile, Dw + Dp), lambda i, ids_ref: (i, 0)),
            scratch_shapes=[pltpu.SemaphoreType.DMA(()),
                            pltpu.SemaphoreType.DMA(())],
        ),
        compiler_params=pltpu.CompilerParams(
            dimension_semantics=("parallel",),
            disable_bounds_checks=True,
        ),
    )(ids, word_table, pos_used)
    return out
